# baseline (speedup 1.0000x reference)
import jax
import jax.numpy as jnp
from jax.experimental import pallas as pl
from jax.experimental.pallas import tpu as pltpu

B = 8192
D = 2048
E = 16
R = 128
TOKEN_TILE = 256
NEG_BIG = -3.0e38


def _bf16_dot(a_f32, w_bf16):
    return jax.lax.dot_general(
        a_f32.astype(jnp.bfloat16), w_bf16, (((1,), (1,)), ((), ())),
        preferred_element_type=jnp.float32)


def _encode_route_body(x_ref, wenc_ref, benc_ref, wgate_ref, enc_ref,
                       comb_ref):
    enc = _bf16_dot(x_ref[...], wenc_ref[...]) + benc_ref[...]
    enc_ref[...] = enc

    logits = _bf16_dot(enc, wgate_ref[...])

    lane = jax.lax.broadcasted_iota(jnp.int32, logits.shape, 1)
    v0 = jnp.max(logits, axis=1, keepdims=True)
    i0 = jnp.min(jnp.where(logits == v0, lane, E), axis=1, keepdims=True)
    masked = jnp.where(lane == i0, NEG_BIG, logits)
    v1 = jnp.max(masked, axis=1, keepdims=True)
    i1 = jnp.min(jnp.where(masked == v1, lane, E), axis=1, keepdims=True)

    e1 = jnp.exp(v1 - v0)
    denom = 1.0 + e1 + 1e-12
    w0 = 1.0 / denom
    w1 = e1 / denom
    w0 = jnp.where(w0 > 1e-12, w0, 0.0)
    w1 = jnp.where(w1 > 1e-12, w1, 0.0)

    comb_ref[...] = (jnp.where(lane == i0, w0, 0.0)
                     + jnp.where(lane == i1, w1, 0.0))


def _experts_body(enc_ref, comb_ref, gamma_ref, u_ref, vt_ref, y_ref):
    enc = enc_ref[...]
    comb = comb_ref[...]
    comb_g = comb * gamma_ref[...]
    h = _bf16_dot(enc, u_ref[...])
    h = h * jax.nn.sigmoid(h)
    h = jnp.concatenate(
        [h[:, m * R:(m + 1) * R] * comb_g[:, m:m + 1] for m in range(E)],
        axis=1)
    o = jax.lax.dot_general(
        h.astype(jnp.bfloat16), vt_ref[...], (((1,), (0,)), ((), ())),
        preferred_element_type=jnp.float32)
    y_ref[...] = enc * jnp.sum(comb, axis=1, keepdims=True) + o


@jax.jit
def kernel(x, W_enc, b_enc, W_gate, U, V, gamma):
    bf16 = jnp.bfloat16
    wenc_b = W_enc.astype(bf16)
    wgate_b = W_gate.astype(bf16)
    u_b = U.reshape(E * R, D).astype(bf16)
    vt_b = V.transpose(0, 2, 1).reshape(E * R, D).astype(bf16)

    grid = (B // TOKEN_TILE,)
    encoded, combine = pl.pallas_call(
        _encode_route_body,
        grid=grid,
        in_specs=[
            pl.BlockSpec((TOKEN_TILE, D), lambda i: (i, 0)),
            pl.BlockSpec((D, D), lambda i: (0, 0)),
            pl.BlockSpec((1, D), lambda i: (0, 0)),
            pl.BlockSpec((E, D), lambda i: (0, 0)),
        ],
        out_specs=[
            pl.BlockSpec((TOKEN_TILE, D), lambda i: (i, 0)),
            pl.BlockSpec((TOKEN_TILE, E), lambda i: (i, 0)),
        ],
        out_shape=[
            jax.ShapeDtypeStruct((B, D), jnp.float32),
            jax.ShapeDtypeStruct((B, E), jnp.float32),
        ],
        compiler_params=pltpu.CompilerParams(
            dimension_semantics=("arbitrary",),
        ),
    )(x, wenc_b, b_enc.reshape(1, D), wgate_b)

    y = pl.pallas_call(
        _experts_body,
        grid=grid,
        in_specs=[
            pl.BlockSpec((TOKEN_TILE, D), lambda i: (i, 0)),
            pl.BlockSpec((TOKEN_TILE, E), lambda i: (i, 0)),
            pl.BlockSpec((1, E), lambda i: (0, 0)),
            pl.BlockSpec((E * R, D), lambda i: (0, 0)),
            pl.BlockSpec((E * R, D), lambda i: (0, 0)),
        ],
        out_specs=pl.BlockSpec((TOKEN_TILE, D), lambda i: (i, 0)),
        out_shape=jax.ShapeDtypeStruct((B, D), jnp.float32),
        compiler_params=pltpu.CompilerParams(
            dimension_semantics=("arbitrary",),
        ),
    )(encoded, combine, gamma.reshape(1, E), u_b, vt_b)
    return y


# speedup vs baseline: 1.1390x; 1.1390x over previous
import jax
import jax.numpy as jnp
from jax.experimental import pallas as pl
from jax.experimental.pallas import tpu as pltpu

B = 8192
D = 2048
E = 16
R = 128
TOKEN_TILE = 512
NEG_BIG = -3.0e38


def _encode_route_body(x_ref, wenc_ref, benc_ref, wgate_ref, enc_ref,
                       comb_ref):
    prec = jax.lax.Precision.DEFAULT
    enc = jax.lax.dot_general(
        x_ref[...], wenc_ref[...], (((1,), (1,)), ((), ())),
        precision=prec, preferred_element_type=jnp.float32)
    enc = enc + benc_ref[...]
    enc_ref[...] = enc

    logits = jax.lax.dot_general(
        enc, wgate_ref[...], (((1,), (1,)), ((), ())),
        precision=prec, preferred_element_type=jnp.float32)

    lane = jax.lax.broadcasted_iota(jnp.int32, logits.shape, 1)
    v0 = jnp.max(logits, axis=1, keepdims=True)
    i0 = jnp.min(jnp.where(logits == v0, lane, E), axis=1, keepdims=True)
    masked = jnp.where(lane == i0, NEG_BIG, logits)
    v1 = jnp.max(masked, axis=1, keepdims=True)
    i1 = jnp.min(jnp.where(masked == v1, lane, E), axis=1, keepdims=True)

    e1 = jnp.exp(v1 - v0)
    denom = 1.0 + e1 + 1e-12
    w0 = 1.0 / denom
    w1 = e1 / denom
    w0 = jnp.where(w0 > 1e-12, w0, 0.0)
    w1 = jnp.where(w1 > 1e-12, w1, 0.0)

    comb_ref[...] = (jnp.where(lane == i0, w0, 0.0)
                     + jnp.where(lane == i1, w1, 0.0))


def _experts_body(enc_ref, comb_ref, gamma_ref, u_ref, vt_ref, y_ref):
    prec = jax.lax.Precision.DEFAULT
    enc = enc_ref[...]
    comb = comb_ref[...]
    comb_g = comb * gamma_ref[...]
    h = jax.lax.dot_general(
        enc, u_ref[...], (((1,), (1,)), ((), ())),
        precision=prec, preferred_element_type=jnp.float32)
    h = h * jax.nn.sigmoid(h)
    h = jnp.concatenate(
        [h[:, m * R:(m + 1) * R] * comb_g[:, m:m + 1] for m in range(E)],
        axis=1)
    o = jax.lax.dot_general(
        h, vt_ref[...], (((1,), (0,)), ((), ())),
        precision=prec, preferred_element_type=jnp.float32)
    y_ref[...] = enc * jnp.sum(comb, axis=1, keepdims=True) + o


@jax.jit
def kernel(x, W_enc, b_enc, W_gate, U, V, gamma):
    grid = (B // TOKEN_TILE,)
    encoded, combine = pl.pallas_call(
        _encode_route_body,
        grid=grid,
        in_specs=[
            pl.BlockSpec((TOKEN_TILE, D), lambda i: (i, 0)),
            pl.BlockSpec((D, D), lambda i: (0, 0)),
            pl.BlockSpec((1, D), lambda i: (0, 0)),
            pl.BlockSpec((E, D), lambda i: (0, 0)),
        ],
        out_specs=[
            pl.BlockSpec((TOKEN_TILE, D), lambda i: (i, 0)),
            pl.BlockSpec((TOKEN_TILE, E), lambda i: (i, 0)),
        ],
        out_shape=[
            jax.ShapeDtypeStruct((B, D), jnp.float32),
            jax.ShapeDtypeStruct((B, E), jnp.float32),
        ],
        compiler_params=pltpu.CompilerParams(
            dimension_semantics=("arbitrary",),
        ),
    )(x, W_enc, b_enc.reshape(1, D), W_gate)

    y = pl.pallas_call(
        _experts_body,
        grid=grid,
        in_specs=[
            pl.BlockSpec((TOKEN_TILE, D), lambda i: (i, 0)),
            pl.BlockSpec((TOKEN_TILE, E), lambda i: (i, 0)),
            pl.BlockSpec((1, E), lambda i: (0, 0)),
            pl.BlockSpec((E * R, D), lambda i: (0, 0)),
            pl.BlockSpec((E * R, D), lambda i: (0, 0)),
        ],
        out_specs=pl.BlockSpec((TOKEN_TILE, D), lambda i: (i, 0)),
        out_shape=jax.ShapeDtypeStruct((B, D), jnp.float32),
        compiler_params=pltpu.CompilerParams(
            dimension_semantics=("arbitrary",),
        ),
    )(encoded, combine, gamma.reshape(1, E), U.reshape(E * R, D),
      V.transpose(0, 2, 1).reshape(E * R, D))
    return y
